# Initial kernel scaffold; baseline (speedup 1.0000x reference)
import jax
import jax.numpy as jnp
from jax.experimental import pallas as pl


def kernel(xyz, color):
    raise NotImplementedError("write your pallas kernel here")


# speedup vs baseline: 3.4459x; 3.4459x over previous
import functools

import jax
import jax.numpy as jnp
from jax import lax
from jax.experimental import pallas as pl
from jax.experimental.pallas import tpu as pltpu
from jax.experimental.pallas import tpu_sc as plsc

NG = 512
KS = 32
B = 32
N = 8192
NR = 64
NC_ = 128


def _fps_body(x_ref, y_ref, z_ref, out_ref, dist_ref):
    x = x_ref[0]
    y = y_ref[0]
    z = z_ref[0]
    row = lax.broadcasted_iota(jnp.int32, (NR, NC_), 0)
    col = lax.broadcasted_iota(jnp.int32, (NR, NC_), 1)
    iota = row * NC_ + col
    dist_ref[...] = jnp.full((NR, NC_), 1e10, dtype=jnp.float32)

    def step(i, far):
        out_ref[0, 0, i] = far
        sel = iota == far
        cx = jnp.sum(jnp.where(sel, x, 0.0))
        cy = jnp.sum(jnp.where(sel, y, 0.0))
        cz = jnp.sum(jnp.where(sel, z, 0.0))
        dx = x - cx
        dy = y - cy
        dz = z - cz
        d = dx * dx + dy * dy + dz * dz
        dmin = jnp.minimum(dist_ref[...], d)
        dist_ref[...] = dmin
        m = jnp.max(dmin)
        nxt = jnp.min(jnp.where(dmin == m, iota, N)).astype(jnp.int32)
        return nxt

    lax.fori_loop(0, NG, step, jnp.int32(0))


def _fps(xyz):
    xt = xyz.transpose(2, 0, 1).reshape(3, B, NR, NC_)
    return pl.pallas_call(
        _fps_body,
        grid=(B,),
        in_specs=[
            pl.BlockSpec((1, NR, NC_), lambda b: (b, 0, 0)),
            pl.BlockSpec((1, NR, NC_), lambda b: (b, 0, 0)),
            pl.BlockSpec((1, NR, NC_), lambda b: (b, 0, 0)),
        ],
        out_specs=pl.BlockSpec((1, 1, NG), lambda b: (b, 0, 0),
                               memory_space=pltpu.SMEM),
        out_shape=jax.ShapeDtypeStruct((B, 1, NG), jnp.int32),
        scratch_shapes=[pltpu.VMEM((NR, NC_), jnp.float32)],
    )(xt[0], xt[1], xt[2]).reshape(B, NG)


NW = 32
TW = 16
_SC_MESH = plsc.VectorSubcoreMesh(core_axis_name="c", subcore_axis_name="s")


def _wid():
    return lax.axis_index("s") * 2 + lax.axis_index("c")


def _center_gather(tab, cidx):
    nrow = B * NG
    per_w = nrow // NW

    @functools.partial(
        pl.kernel, mesh=_SC_MESH,
        compiler_params=pltpu.CompilerParams(use_tc_tiling_on_sc=False),
        out_type=jax.ShapeDtypeStruct((nrow, TW), jnp.float32),
        scratch_types=[
            pltpu.VMEM((per_w,), jnp.int32),
            pltpu.VMEM((per_w, TW), jnp.float32),
            pltpu.SemaphoreType.DMA,
        ],
    )
    def k(tab_hbm, cidx_hbm, out_hbm, idx_v, rows_v, sem):
        base = _wid() * per_w
        pltpu.sync_copy(cidx_hbm.at[pl.ds(base, per_w)], idx_v)
        pltpu.async_copy(tab_hbm.at[idx_v], rows_v, sem).wait()
        pltpu.sync_copy(rows_v, out_hbm.at[pl.ds(base, per_w)])

    return k(tab, cidx)


def _nbr_gather(tab, nidx, crows):
    nrow = B * NG * KS
    per_w = nrow // NW
    nchunk = 4
    chunk = per_w // nchunk
    gchunk = chunk // KS
    cg_per_w = B * NG // NW

    @functools.partial(
        pl.kernel, mesh=_SC_MESH,
        compiler_params=pltpu.CompilerParams(use_tc_tiling_on_sc=False),
        out_type=jax.ShapeDtypeStruct((nrow, TW), jnp.float32),
        scratch_types=[
            pltpu.VMEM((chunk,), jnp.int32),
            pltpu.VMEM((chunk, TW), jnp.float32),
            pltpu.VMEM((cg_per_w, TW), jnp.float32),
            pltpu.SemaphoreType.DMA,
        ],
    )
    def k(tab_hbm, nidx_hbm, crows_hbm, out_hbm, idx_v, rows_v, crows_v, sem):
        wid = _wid()
        rbase = wid * per_w
        pltpu.sync_copy(crows_hbm.at[pl.ds(wid * cg_per_w, cg_per_w)], crows_v)
        lane = lax.broadcasted_iota(jnp.int32, (TW,), 0)
        for c in range(nchunk):
            pltpu.sync_copy(nidx_hbm.at[pl.ds(rbase + c * chunk, chunk)], idx_v)
            pltpu.async_copy(tab_hbm.at[idx_v], rows_v, sem).wait()

            def group_body(g, _):
                crow = crows_v[c * gchunk + g, :]
                csub = jnp.where(lane < 3, crow, 0.0)

                def row_body(r, _):
                    rows_v[g * KS + r, :] = rows_v[g * KS + r, :] - csub
                    return 0

                return lax.fori_loop(0, KS, row_body, 0)

            lax.fori_loop(0, gchunk, group_body, 0)
            pltpu.sync_copy(rows_v, out_hbm.at[pl.ds(rbase + c * chunk, chunk)])

    return k(tab, nidx, crows)


_INF = jnp.float32(1e30)


def _bfround(x):
    u = plsc.bitcast(x, jnp.uint32)
    r = (u + 0x7FFF + ((u >> 16) & 1)) & jnp.uint32(0xFFFF0000)
    return plsc.bitcast(r, jnp.float32)


def _sort16(k, v):
    return plsc.sort_key_val(k, v)


def _merge32(a0k, a0v, a1k, a1v, ck, cv):
    crk = lax.rev(ck, (0,))
    crv = lax.rev(cv, (0,))
    lt = crk < a1k
    l1k = jnp.where(lt, crk, a1k)
    l1v = jnp.where(lt, crv, a1v)
    h1k = jnp.where(lt, a1k, crk)
    lt2 = l1k < a0k
    m0k = jnp.where(lt2, l1k, a0k)
    m0v = jnp.where(lt2, l1v, a0v)
    m1k = jnp.where(lt2, a0k, l1k)
    m1v = jnp.where(lt2, a0v, l1v)
    b0k, b0v = _sort16(m0k, m0v)
    b1k, b1v = _sort16(m1k, m1v)
    return b0k, b0v, b1k, b1v, jnp.min(h1k)


def _knn_topk(xt, crows):
    nrow = B * NG
    g_per_w = nrow // NW
    nchunk = N // 16

    @functools.partial(
        pl.kernel, mesh=_SC_MESH,
        compiler_params=pltpu.CompilerParams(use_tc_tiling_on_sc=False,
                                             needs_layout_passes=False),
        out_type=jax.ShapeDtypeStruct((nrow, KS), jnp.int32),
        scratch_types=[
            pltpu.VMEM((N,), jnp.float32),
            pltpu.VMEM((N,), jnp.float32),
            pltpu.VMEM((N,), jnp.float32),
            pltpu.VMEM((N,), jnp.float32),
            pltpu.VMEM((g_per_w, TW), jnp.float32),
            pltpu.VMEM((g_per_w, KS), jnp.int32),
            pltpu.VMEM((48,), jnp.float32),
            pltpu.SemaphoreType.DMA,
        ],
    )
    def k(xt_hbm, crows_hbm, out_hbm, x_v, y_v, z_v, n2_v, crows_v, idxb_v,
          kbuf_v, sem):
        wid = _wid()
        pltpu.sync_copy(xt_hbm.at[0, wid], x_v)
        pltpu.sync_copy(xt_hbm.at[1, wid], y_v)
        pltpu.sync_copy(xt_hbm.at[2, wid], z_v)
        pltpu.sync_copy(crows_hbm.at[pl.ds(wid * g_per_w, g_per_w)], crows_v)

        def norm_body(j, _):
            xv = x_v[pl.ds(j * 16, 16)]
            yv = y_v[pl.ds(j * 16, 16)]
            zv = z_v[pl.ds(j * 16, 16)]
            n2_v[pl.ds(j * 16, 16)] = xv * xv + yv * yv + zv * zv
            x_v[pl.ds(j * 16, 16)] = _bfround(xv)
            y_v[pl.ds(j * 16, 16)] = _bfround(yv)
            z_v[pl.ds(j * 16, 16)] = _bfround(zv)
            return 0

        lax.fori_loop(0, nchunk, norm_body, 0)

        iota16 = lax.broadcasted_iota(jnp.int32, (16,), 0)
        zeros16 = jnp.zeros((16,), jnp.int32)
        kbuf_v[pl.ds(32, 16)] = jnp.full((16,), _INF, jnp.float32)

        def row_body(g, _):
            gs = jnp.full((16,), g, jnp.int32)
            cx = plsc.load_gather(crows_v, [gs, zeros16])
            cy = plsc.load_gather(crows_v, [gs, zeros16 + 1])
            cz = plsc.load_gather(crows_v, [gs, zeros16 + 2])
            cxn = _bfround(cx) * -2.0
            cyn = _bfround(cy) * -2.0
            czn = _bfround(cz) * -2.0
            nc2 = cx * cx + cy * cy + cz * cz

            def dchunk(j):
                xv = x_v[pl.ds(j * 16, 16)]
                yv = y_v[pl.ds(j * 16, 16)]
                zv = z_v[pl.ds(j * 16, 16)]
                n2 = n2_v[pl.ds(j * 16, 16)]
                t = xv * cxn + yv * cyn + zv * czn
                d = (t + nc2) + n2
                return d, iota16 + j * 16

            d0, i0 = dchunk(0)
            d1, i1 = dchunk(1)
            s0k, s0v = _sort16(d0, i0)
            s1k, s1v = _sort16(d1, i1)
            r1k = lax.rev(s1k, (0,))
            r1v = lax.rev(s1v, (0,))
            lt = r1k < s0k
            lk = jnp.where(lt, r1k, s0k)
            lv = jnp.where(lt, r1v, s0v)
            hk = jnp.where(lt, s0k, r1k)
            hv = jnp.where(lt, s0v, r1v)
            a0k, a0v = _sort16(lk, lv)
            a1k, a1v = _sort16(hk, hv)
            tau = jnp.max(a1k)

            def chunk_body(j, st):
                b0k, b0v, b1k, b1v, t, fl = st
                dj, ij = dchunk(j)
                mask = dj < t
                fl = fl | jnp.any(dj == t)

                def do_merge(op):
                    c0k, c0v, c1k, c1v, _, f = op
                    dm = jnp.where(mask, dj, _INF)
                    im = jnp.where(mask, ij, 0)
                    sck, scv = _sort16(dm, im)
                    n0k, n0v, n1k, n1v, evmin = _merge32(
                        c0k, c0v, c1k, c1v, sck, scv)
                    nt = jnp.max(n1k)
                    return n0k, n0v, n1k, n1v, nt, f | (evmin == nt)

                return lax.cond(jnp.any(mask), do_merge, lambda op: op,
                                (b0k, b0v, b1k, b1v, t, fl))

            a0k, a0v, a1k, a1v, tau, flag = lax.fori_loop(
                2, nchunk, chunk_body,
                (a0k, a0v, a1k, a1v, tau, jnp.bool_(False)))

            kbuf_v[pl.ds(0, 16)] = a0k
            kbuf_v[pl.ds(16, 16)] = a1k
            g1 = plsc.load_gather(kbuf_v, [iota16 + 1])
            g2 = plsc.load_gather(kbuf_v, [iota16 + 17])
            tied = flag | jnp.any(a0k == g1) | jnp.any(a1k == g2)

            def repair(_):
                bigi = jnp.int32(2 ** 30)
                o0 = jnp.zeros((16,), jnp.int32)
                o1 = jnp.zeros((16,), jnp.int32)
                lastk = jnp.float32(-1e30)
                lasti = jnp.int32(-1)
                for s in range(KS):
                    def scan_chunk(j, acc):
                        bk, bi = acc
                        dj, ij = dchunk(j)
                        m = (dj > lastk) | ((dj == lastk) & (ij > lasti))
                        ck = jnp.where(m, dj, _INF)
                        ci = jnp.where(m, ij, bigi)
                        lt = (ck < bk) | ((ck == bk) & (ci < bi))
                        return (jnp.where(lt, ck, bk), jnp.where(lt, ci, bi))

                    bk, bi = lax.fori_loop(
                        0, nchunk, scan_chunk,
                        (jnp.full((16,), _INF, jnp.float32),
                         jnp.full((16,), bigi, jnp.int32)))
                    mk = jnp.min(bk)
                    mi = jnp.min(jnp.where(bk == mk, bi, bigi))
                    lastk = mk
                    lasti = mi
                    if s < 16:
                        o0 = jnp.where(iota16 == s, mi, o0)
                    else:
                        o1 = jnp.where(iota16 == (s - 16), mi, o1)
                return o0, o1

            o0, o1 = lax.cond(tied, repair, lambda _: (a0v, a1v), 0)
            idxb_v[g, pl.ds(0, 16)] = o0
            idxb_v[g, pl.ds(16, 16)] = o1
            return 0

        lax.fori_loop(0, g_per_w, row_body, 0)
        pltpu.sync_copy(idxb_v, out_hbm.at[pl.ds(wid * g_per_w, g_per_w)])

    return k(xt, crows)


def kernel(xyz, color):
    fps_idx = _fps(xyz)

    tab = jnp.concatenate(
        [xyz, color, jnp.zeros((B, N, TW - 9), jnp.float32)], axis=-1
    ).reshape(B * N, TW)
    bbase = (jnp.arange(B, dtype=jnp.int32) * N)[:, None]
    cidx = (fps_idx + bbase).reshape(-1)
    crows = _center_gather(tab, cidx)
    center = crows.reshape(B, NG, TW)[..., :3]

    xt = xyz.transpose(2, 0, 1)
    idx = _knn_topk(xt, crows).reshape(B, NG, KS)

    nidx = (idx + bbase[:, :, None]).reshape(-1)
    nrows = _nbr_gather(tab, nidx, crows).reshape(B, NG, KS, TW)
    neighborhood = nrows[..., :3]
    features = nrows[..., :9]
    return (neighborhood, center, features)


# speedup vs baseline: 4.8584x; 1.4099x over previous
import functools

import jax
import jax.numpy as jnp
from jax import lax
from jax.experimental import pallas as pl
from jax.experimental.pallas import tpu as pltpu
from jax.experimental.pallas import tpu_sc as plsc

NG = 512
KS = 32
B = 32
N = 8192
NR = 64
NC_ = 128


def _fps_step(xt_ref, out_ref, dist_ref, far_ref):
    i = pl.program_id(0)
    x = xt_ref[0]
    y = xt_ref[1]
    z = xt_ref[2]

    @pl.when(i == 0)
    def _():
        dist_ref[...] = jnp.full((B, N), 1e10, dtype=jnp.float32)
        far_ref[...] = jnp.zeros((B, 1), jnp.int32)

    far = far_ref[...]
    out_ref[0] = far
    iota = lax.broadcasted_iota(jnp.int32, (B, N), 1)
    sel = iota == far
    cx = jnp.sum(jnp.where(sel, x, 0.0), axis=1, keepdims=True)
    cy = jnp.sum(jnp.where(sel, y, 0.0), axis=1, keepdims=True)
    cz = jnp.sum(jnp.where(sel, z, 0.0), axis=1, keepdims=True)
    dx = x - cx
    dy = y - cy
    dz = z - cz
    d = dx * dx + dy * dy + dz * dz
    dmin = jnp.minimum(dist_ref[...], d)
    dist_ref[...] = dmin
    m = jnp.max(dmin, axis=1, keepdims=True)
    far_ref[...] = jnp.min(jnp.where(dmin == m, iota, N), axis=1,
                           keepdims=True).astype(jnp.int32)


def _fps(xt):
    out = pl.pallas_call(
        _fps_step,
        grid=(NG,),
        in_specs=[pl.BlockSpec((3, B, N), lambda i: (0, 0, 0))],
        out_specs=pl.BlockSpec((1, B, 1), lambda i: (i, 0, 0)),
        out_shape=jax.ShapeDtypeStruct((NG, B, 1), jnp.int32),
        scratch_shapes=[pltpu.VMEM((B, N), jnp.float32),
                        pltpu.VMEM((B, 1), jnp.int32)],
    )(xt)
    return out.reshape(NG, B).T


NW = 32
TW = 16
_SC_MESH = plsc.VectorSubcoreMesh(core_axis_name="c", subcore_axis_name="s")


def _wid():
    return lax.axis_index("s") * 2 + lax.axis_index("c")


def _center_gather(tab, cidx):
    nrow = B * NG
    per_w = nrow // NW

    @functools.partial(
        pl.kernel, mesh=_SC_MESH,
        compiler_params=pltpu.CompilerParams(use_tc_tiling_on_sc=False),
        out_type=jax.ShapeDtypeStruct((nrow, TW), jnp.float32),
        scratch_types=[
            pltpu.VMEM((per_w,), jnp.int32),
            pltpu.VMEM((per_w, TW), jnp.float32),
            pltpu.SemaphoreType.DMA,
        ],
    )
    def k(tab_hbm, cidx_hbm, out_hbm, idx_v, rows_v, sem):
        base = _wid() * per_w
        pltpu.sync_copy(cidx_hbm.at[pl.ds(base, per_w)], idx_v)
        pltpu.async_copy(tab_hbm.at[idx_v], rows_v, sem).wait()
        pltpu.sync_copy(rows_v, out_hbm.at[pl.ds(base, per_w)])

    return k(tab, cidx)


def _nbr_gather(tab, nidx, crows):
    nrow = B * NG * KS
    per_w = nrow // NW
    nchunk = 4
    chunk = per_w // nchunk
    gchunk = chunk // KS
    cg_per_w = B * NG // NW

    @functools.partial(
        pl.kernel, mesh=_SC_MESH,
        compiler_params=pltpu.CompilerParams(use_tc_tiling_on_sc=False),
        out_type=jax.ShapeDtypeStruct((nrow, TW), jnp.float32),
        scratch_types=[
            pltpu.VMEM((chunk,), jnp.int32),
            pltpu.VMEM((chunk, TW), jnp.float32),
            pltpu.VMEM((cg_per_w, TW), jnp.float32),
            pltpu.SemaphoreType.DMA,
        ],
    )
    def k(tab_hbm, nidx_hbm, crows_hbm, out_hbm, idx_v, rows_v, crows_v, sem):
        wid = _wid()
        rbase = wid * per_w
        pltpu.sync_copy(crows_hbm.at[pl.ds(wid * cg_per_w, cg_per_w)], crows_v)
        lane = lax.broadcasted_iota(jnp.int32, (TW,), 0)
        for c in range(nchunk):
            pltpu.sync_copy(nidx_hbm.at[pl.ds(rbase + c * chunk, chunk)], idx_v)
            pltpu.async_copy(tab_hbm.at[idx_v], rows_v, sem).wait()

            def group_body(g, _):
                crow = crows_v[c * gchunk + g, :]
                csub = jnp.where(lane < 3, crow, 0.0)

                def row_body(r, _):
                    rows_v[g * KS + r, :] = rows_v[g * KS + r, :] - csub
                    return 0

                return lax.fori_loop(0, KS, row_body, 0)

            lax.fori_loop(0, gchunk, group_body, 0)
            pltpu.sync_copy(rows_v, out_hbm.at[pl.ds(rbase + c * chunk, chunk)])

    return k(tab, nidx, crows)


_INF = jnp.float32(1e30)


def _bfround(x):
    u = plsc.bitcast(x, jnp.uint32)
    r = (u + 0x7FFF + ((u >> 16) & 1)) & jnp.uint32(0xFFFF0000)
    return plsc.bitcast(r, jnp.float32)


def _sort16(k, v):
    return plsc.sort_key_val(k, v)


def _merge32(a0k, a0v, a1k, a1v, ck, cv):
    crk = lax.rev(ck, (0,))
    crv = lax.rev(cv, (0,))
    lt = crk < a1k
    l1k = jnp.where(lt, crk, a1k)
    l1v = jnp.where(lt, crv, a1v)
    h1k = jnp.where(lt, a1k, crk)
    lt2 = l1k < a0k
    m0k = jnp.where(lt2, l1k, a0k)
    m0v = jnp.where(lt2, l1v, a0v)
    m1k = jnp.where(lt2, a0k, l1k)
    m1v = jnp.where(lt2, a0v, l1v)
    b0k, b0v = _sort16(m0k, m0v)
    b1k, b1v = _sort16(m1k, m1v)
    return b0k, b0v, b1k, b1v, jnp.min(h1k)


def _knn_topk(xt, crows):
    nrow = B * NG
    g_per_w = nrow // NW
    nchunk = N // 16

    @functools.partial(
        pl.kernel, mesh=_SC_MESH,
        compiler_params=pltpu.CompilerParams(use_tc_tiling_on_sc=False,
                                             needs_layout_passes=False),
        out_type=jax.ShapeDtypeStruct((nrow, KS), jnp.int32),
        scratch_types=[
            pltpu.VMEM((N,), jnp.float32),
            pltpu.VMEM((N,), jnp.float32),
            pltpu.VMEM((N,), jnp.float32),
            pltpu.VMEM((N,), jnp.float32),
            pltpu.VMEM((g_per_w, TW), jnp.float32),
            pltpu.VMEM((g_per_w, KS), jnp.int32),
            pltpu.VMEM((48,), jnp.float32),
            pltpu.SemaphoreType.DMA,
        ],
    )
    def k(xt_hbm, crows_hbm, out_hbm, x_v, y_v, z_v, n2_v, crows_v, idxb_v,
          kbuf_v, sem):
        wid = _wid()
        pltpu.sync_copy(xt_hbm.at[0, wid], x_v)
        pltpu.sync_copy(xt_hbm.at[1, wid], y_v)
        pltpu.sync_copy(xt_hbm.at[2, wid], z_v)
        pltpu.sync_copy(crows_hbm.at[pl.ds(wid * g_per_w, g_per_w)], crows_v)

        def norm_body(j, _):
            xv = x_v[pl.ds(j * 16, 16)]
            yv = y_v[pl.ds(j * 16, 16)]
            zv = z_v[pl.ds(j * 16, 16)]
            n2_v[pl.ds(j * 16, 16)] = xv * xv + yv * yv + zv * zv
            x_v[pl.ds(j * 16, 16)] = _bfround(xv)
            y_v[pl.ds(j * 16, 16)] = _bfround(yv)
            z_v[pl.ds(j * 16, 16)] = _bfround(zv)
            return 0

        lax.fori_loop(0, nchunk, norm_body, 0)

        iota16 = lax.broadcasted_iota(jnp.int32, (16,), 0)
        zeros16 = jnp.zeros((16,), jnp.int32)
        kbuf_v[pl.ds(32, 16)] = jnp.full((16,), _INF, jnp.float32)

        def row_body(g, _):
            gs = jnp.full((16,), g, jnp.int32)
            cx = plsc.load_gather(crows_v, [gs, zeros16])
            cy = plsc.load_gather(crows_v, [gs, zeros16 + 1])
            cz = plsc.load_gather(crows_v, [gs, zeros16 + 2])
            cxn = _bfround(cx) * -2.0
            cyn = _bfround(cy) * -2.0
            czn = _bfround(cz) * -2.0
            nc2 = cx * cx + cy * cy + cz * cz

            def dchunk(j):
                xv = x_v[pl.ds(j * 16, 16)]
                yv = y_v[pl.ds(j * 16, 16)]
                zv = z_v[pl.ds(j * 16, 16)]
                n2 = n2_v[pl.ds(j * 16, 16)]
                t = xv * cxn + yv * cyn + zv * czn
                d = (t + nc2) + n2
                return d, iota16 + j * 16

            d0, i0 = dchunk(0)
            d1, i1 = dchunk(1)
            s0k, s0v = _sort16(d0, i0)
            s1k, s1v = _sort16(d1, i1)
            r1k = lax.rev(s1k, (0,))
            r1v = lax.rev(s1v, (0,))
            lt = r1k < s0k
            lk = jnp.where(lt, r1k, s0k)
            lv = jnp.where(lt, r1v, s0v)
            hk = jnp.where(lt, s0k, r1k)
            hv = jnp.where(lt, s0v, r1v)
            a0k, a0v = _sort16(lk, lv)
            a1k, a1v = _sort16(hk, hv)
            tau = jnp.max(a1k)

            def chunk_body(j, st):
                b0k, b0v, b1k, b1v, t, fl = st
                dj, ij = dchunk(j)
                mask = dj < t
                fl = fl | jnp.any(dj == t)

                def do_merge(op):
                    c0k, c0v, c1k, c1v, _, f = op
                    dm = jnp.where(mask, dj, _INF)
                    im = jnp.where(mask, ij, 0)
                    sck, scv = _sort16(dm, im)
                    n0k, n0v, n1k, n1v, evmin = _merge32(
                        c0k, c0v, c1k, c1v, sck, scv)
                    nt = jnp.max(n1k)
                    return n0k, n0v, n1k, n1v, nt, f | (evmin == nt)

                return lax.cond(jnp.any(mask), do_merge, lambda op: op,
                                (b0k, b0v, b1k, b1v, t, fl))

            a0k, a0v, a1k, a1v, tau, flag = lax.fori_loop(
                2, nchunk, chunk_body,
                (a0k, a0v, a1k, a1v, tau, jnp.bool_(False)))

            kbuf_v[pl.ds(0, 16)] = a0k
            kbuf_v[pl.ds(16, 16)] = a1k
            g1 = plsc.load_gather(kbuf_v, [iota16 + 1])
            g2 = plsc.load_gather(kbuf_v, [iota16 + 17])
            tied = flag | jnp.any(a0k == g1) | jnp.any(a1k == g2)

            def repair(_):
                bigi = jnp.int32(2 ** 30)
                o0 = jnp.zeros((16,), jnp.int32)
                o1 = jnp.zeros((16,), jnp.int32)
                lastk = jnp.float32(-1e30)
                lasti = jnp.int32(-1)
                for s in range(KS):
                    def scan_chunk(j, acc):
                        bk, bi = acc
                        dj, ij = dchunk(j)
                        m = (dj > lastk) | ((dj == lastk) & (ij > lasti))
                        ck = jnp.where(m, dj, _INF)
                        ci = jnp.where(m, ij, bigi)
                        lt = (ck < bk) | ((ck == bk) & (ci < bi))
                        return (jnp.where(lt, ck, bk), jnp.where(lt, ci, bi))

                    bk, bi = lax.fori_loop(
                        0, nchunk, scan_chunk,
                        (jnp.full((16,), _INF, jnp.float32),
                         jnp.full((16,), bigi, jnp.int32)))
                    mk = jnp.min(bk)
                    mi = jnp.min(jnp.where(bk == mk, bi, bigi))
                    lastk = mk
                    lasti = mi
                    if s < 16:
                        o0 = jnp.where(iota16 == s, mi, o0)
                    else:
                        o1 = jnp.where(iota16 == (s - 16), mi, o1)
                return o0, o1

            o0, o1 = lax.cond(tied, repair, lambda _: (a0v, a1v), 0)
            idxb_v[g, pl.ds(0, 16)] = o0
            idxb_v[g, pl.ds(16, 16)] = o1
            return 0

        lax.fori_loop(0, g_per_w, row_body, 0)
        pltpu.sync_copy(idxb_v, out_hbm.at[pl.ds(wid * g_per_w, g_per_w)])

    return k(xt, crows)


def kernel(xyz, color):
    xt = xyz.transpose(2, 0, 1)
    fps_idx = _fps(xt)

    tab = jnp.concatenate(
        [xyz, color, jnp.zeros((B, N, TW - 9), jnp.float32)], axis=-1
    ).reshape(B * N, TW)
    bbase = (jnp.arange(B, dtype=jnp.int32) * N)[:, None]
    cidx = (fps_idx + bbase).reshape(-1)
    crows = _center_gather(tab, cidx)
    center = crows.reshape(B, NG, TW)[..., :3]

    idx = _knn_topk(xt, crows).reshape(B, NG, KS)

    nidx = (idx + bbase[:, :, None]).reshape(-1)
    nrows = _nbr_gather(tab, nidx, crows).reshape(B, NG, KS, TW)
    neighborhood = nrows[..., :3]
    features = nrows[..., :9]
    return (neighborhood, center, features)


# speedup vs baseline: 7.7666x; 1.5986x over previous
import functools

import jax
import jax.numpy as jnp
from jax import lax
from jax.experimental import pallas as pl
from jax.experimental.pallas import tpu as pltpu
from jax.experimental.pallas import tpu_sc as plsc

NG = 512
KS = 32
B = 32
N = 8192
NR = 64
NC_ = 128


def _fps_step(xt_ref, out_ref, dist_ref, far_ref):
    i = pl.program_id(0)
    x = xt_ref[0]
    y = xt_ref[1]
    z = xt_ref[2]

    @pl.when(i == 0)
    def _():
        dist_ref[...] = jnp.full((B, N), 1e10, dtype=jnp.float32)
        far_ref[...] = jnp.zeros((B, 1), jnp.int32)

    far = far_ref[...]
    out_ref[0] = far
    iota = lax.broadcasted_iota(jnp.int32, (B, N), 1)
    sel = iota == far
    cx = jnp.sum(jnp.where(sel, x, 0.0), axis=1, keepdims=True)
    cy = jnp.sum(jnp.where(sel, y, 0.0), axis=1, keepdims=True)
    cz = jnp.sum(jnp.where(sel, z, 0.0), axis=1, keepdims=True)
    dx = x - cx
    dy = y - cy
    dz = z - cz
    d = dx * dx + dy * dy + dz * dz
    dmin = jnp.minimum(dist_ref[...], d)
    dist_ref[...] = dmin
    m = jnp.max(dmin, axis=1, keepdims=True)
    far_ref[...] = jnp.min(jnp.where(dmin == m, iota, N), axis=1,
                           keepdims=True).astype(jnp.int32)


def _fps(xt):
    out = pl.pallas_call(
        _fps_step,
        grid=(NG,),
        in_specs=[pl.BlockSpec((3, B, N), lambda i: (0, 0, 0))],
        out_specs=pl.BlockSpec((1, B, 1), lambda i: (i, 0, 0)),
        out_shape=jax.ShapeDtypeStruct((NG, B, 1), jnp.int32),
        scratch_shapes=[pltpu.VMEM((B, N), jnp.float32),
                        pltpu.VMEM((B, 1), jnp.int32)],
    )(xt)
    return out.reshape(NG, B).T


NW = 32
TW = 16
_SC_MESH = plsc.VectorSubcoreMesh(core_axis_name="c", subcore_axis_name="s")


def _wid():
    return lax.axis_index("s") * 2 + lax.axis_index("c")


def _center_gather(tab, cidx):
    nrow = B * NG
    per_w = nrow // NW

    @functools.partial(
        pl.kernel, mesh=_SC_MESH,
        compiler_params=pltpu.CompilerParams(use_tc_tiling_on_sc=False),
        out_type=jax.ShapeDtypeStruct((nrow, TW), jnp.float32),
        scratch_types=[
            pltpu.VMEM((per_w,), jnp.int32),
            pltpu.VMEM((per_w, TW), jnp.float32),
            pltpu.SemaphoreType.DMA,
        ],
    )
    def k(tab_hbm, cidx_hbm, out_hbm, idx_v, rows_v, sem):
        base = _wid() * per_w
        pltpu.sync_copy(cidx_hbm.at[pl.ds(base, per_w)], idx_v)
        pltpu.async_copy(tab_hbm.at[idx_v], rows_v, sem).wait()
        pltpu.sync_copy(rows_v, out_hbm.at[pl.ds(base, per_w)])

    return k(tab, cidx)


def _nbr_gather(tab, nidx, crows):
    nrow = B * NG * KS
    per_w = nrow // NW
    nchunk = 4
    chunk = per_w // nchunk
    gchunk = chunk // KS
    cg_per_w = B * NG // NW

    @functools.partial(
        pl.kernel, mesh=_SC_MESH,
        compiler_params=pltpu.CompilerParams(use_tc_tiling_on_sc=False),
        out_type=jax.ShapeDtypeStruct((nrow, TW), jnp.float32),
        scratch_types=[
            pltpu.VMEM((chunk,), jnp.int32),
            pltpu.VMEM((chunk, TW), jnp.float32),
            pltpu.VMEM((cg_per_w, TW), jnp.float32),
            pltpu.SemaphoreType.DMA,
        ],
    )
    def k(tab_hbm, nidx_hbm, crows_hbm, out_hbm, idx_v, rows_v, crows_v, sem):
        wid = _wid()
        rbase = wid * per_w
        pltpu.sync_copy(crows_hbm.at[pl.ds(wid * cg_per_w, cg_per_w)], crows_v)
        lane = lax.broadcasted_iota(jnp.int32, (TW,), 0)
        for c in range(nchunk):
            pltpu.sync_copy(nidx_hbm.at[pl.ds(rbase + c * chunk, chunk)], idx_v)
            pltpu.async_copy(tab_hbm.at[idx_v], rows_v, sem).wait()

            def group_body(g, _):
                crow = crows_v[c * gchunk + g, :]
                csub = jnp.where(lane < 3, crow, 0.0)

                def row_body(r, _):
                    rows_v[g * KS + r, :] = rows_v[g * KS + r, :] - csub
                    return 0

                return lax.fori_loop(0, KS, row_body, 0)

            lax.fori_loop(0, gchunk, group_body, 0)
            pltpu.sync_copy(rows_v, out_hbm.at[pl.ds(rbase + c * chunk, chunk)])

    return k(tab, nidx, crows)


_INF = jnp.float32(1e30)


def _bfround(x):
    u = plsc.bitcast(x, jnp.uint32)
    r = (u + 0x7FFF + ((u >> 16) & 1)) & jnp.uint32(0xFFFF0000)
    return plsc.bitcast(r, jnp.float32)


def _sort16(k, v):
    return plsc.sort_key_val(k, v)


def _merge32(a0k, a0v, a1k, a1v, ck, cv):
    crk = lax.rev(ck, (0,))
    crv = lax.rev(cv, (0,))
    lt = crk < a1k
    l1k = jnp.where(lt, crk, a1k)
    l1v = jnp.where(lt, crv, a1v)
    h1k = jnp.where(lt, a1k, crk)
    lt2 = l1k < a0k
    m0k = jnp.where(lt2, l1k, a0k)
    m0v = jnp.where(lt2, l1v, a0v)
    m1k = jnp.where(lt2, a0k, l1k)
    m1v = jnp.where(lt2, a0v, l1v)
    b0k, b0v = _sort16(m0k, m0v)
    b1k, b1v = _sort16(m1k, m1v)
    return b0k, b0v, b1k, b1v, jnp.min(h1k)


def _knn_topk(xt, crows):
    nrow = B * NG
    g_per_w = nrow // NW
    nchunk = N // 16

    @functools.partial(
        pl.kernel, mesh=_SC_MESH,
        compiler_params=pltpu.CompilerParams(use_tc_tiling_on_sc=False,
                                             needs_layout_passes=False),
        out_type=jax.ShapeDtypeStruct((nrow, KS), jnp.int32),
        scratch_types=[
            pltpu.VMEM((N,), jnp.float32),
            pltpu.VMEM((N,), jnp.float32),
            pltpu.VMEM((N,), jnp.float32),
            pltpu.VMEM((N,), jnp.float32),
            pltpu.VMEM((g_per_w, TW), jnp.float32),
            pltpu.VMEM((g_per_w, KS), jnp.int32),
            pltpu.VMEM((48,), jnp.float32),
            pltpu.SemaphoreType.DMA,
        ],
    )
    def k(xt_hbm, crows_hbm, out_hbm, x_v, y_v, z_v, n2_v, crows_v, idxb_v,
          kbuf_v, sem):
        wid = _wid()
        pltpu.sync_copy(xt_hbm.at[0, wid], x_v)
        pltpu.sync_copy(xt_hbm.at[1, wid], y_v)
        pltpu.sync_copy(xt_hbm.at[2, wid], z_v)
        pltpu.sync_copy(crows_hbm.at[pl.ds(wid * g_per_w, g_per_w)], crows_v)

        def norm_body(j, _):
            xv = x_v[pl.ds(j * 16, 16)]
            yv = y_v[pl.ds(j * 16, 16)]
            zv = z_v[pl.ds(j * 16, 16)]
            n2_v[pl.ds(j * 16, 16)] = xv * xv + yv * yv + zv * zv
            x_v[pl.ds(j * 16, 16)] = _bfround(xv)
            y_v[pl.ds(j * 16, 16)] = _bfround(yv)
            z_v[pl.ds(j * 16, 16)] = _bfround(zv)
            return 0

        lax.fori_loop(0, nchunk, norm_body, 0)

        iota16 = lax.broadcasted_iota(jnp.int32, (16,), 0)
        zeros16 = jnp.zeros((16,), jnp.int32)
        kbuf_v[pl.ds(32, 16)] = jnp.full((16,), _INF, jnp.float32)

        def row_body(g, _):
            gs = jnp.full((16,), g, jnp.int32)
            cx = plsc.load_gather(crows_v, [gs, zeros16])
            cy = plsc.load_gather(crows_v, [gs, zeros16 + 1])
            cz = plsc.load_gather(crows_v, [gs, zeros16 + 2])
            cxn = _bfround(cx) * -2.0
            cyn = _bfround(cy) * -2.0
            czn = _bfround(cz) * -2.0
            nc2 = cx * cx + cy * cy + cz * cz

            def dchunk(j):
                xv = x_v[pl.ds(j * 16, 16)]
                yv = y_v[pl.ds(j * 16, 16)]
                zv = z_v[pl.ds(j * 16, 16)]
                n2 = n2_v[pl.ds(j * 16, 16)]
                t = xv * cxn + yv * cyn + zv * czn
                d = (t + nc2) + n2
                return d, iota16 + j * 16

            d0, i0 = dchunk(0)
            d1, i1 = dchunk(1)
            s0k, s0v = _sort16(d0, i0)
            s1k, s1v = _sort16(d1, i1)
            r1k = lax.rev(s1k, (0,))
            r1v = lax.rev(s1v, (0,))
            lt = r1k < s0k
            lk = jnp.where(lt, r1k, s0k)
            lv = jnp.where(lt, r1v, s0v)
            hk = jnp.where(lt, s0k, r1k)
            hv = jnp.where(lt, s0v, r1v)
            a0k, a0v = _sort16(lk, lv)
            a1k, a1v = _sort16(hk, hv)
            tau = jnp.max(a1k)

            def merge_one(st, dj, ij, mask):
                def do_merge(op):
                    c0k, c0v, c1k, c1v, t, f = op
                    dm = jnp.where(mask, dj, _INF)
                    im = jnp.where(mask, ij, 0)
                    sck, scv = _sort16(dm, im)
                    n0k, n0v, n1k, n1v, evmin = _merge32(
                        c0k, c0v, c1k, c1v, sck, scv)
                    nt = jnp.max(n1k)
                    return n0k, n0v, n1k, n1v, nt, f | (evmin == nt)

                return lax.cond(jnp.any(mask), do_merge, lambda op: op, st)

            def merge_step(st, j):
                dj, ij = dchunk(j)
                mask = dj < st[4]
                st = (st[0], st[1], st[2], st[3], st[4],
                      st[5] | jnp.any(dj == st[4]))
                return merge_one(st, dj, ij, mask)

            st = (a0k, a0v, a1k, a1v, tau, jnp.bool_(False))
            st = merge_step(st, 2)
            st = merge_step(st, 3)

            def blk_body(bi, st):
                j0 = 4 + bi * 4
                t = st[4]
                dv = [dchunk(j0 + s) for s in range(4)]
                ms = [d < t for d, _ in dv]
                eq = ((dv[0][0] == t) | (dv[1][0] == t)) | \
                     ((dv[2][0] == t) | (dv[3][0] == t))
                st = (st[0], st[1], st[2], st[3], st[4], st[5] | jnp.any(eq))
                hit = jnp.any((ms[0] | ms[1]) | (ms[2] | ms[3]))

                def do_hits(op):
                    for s in range(4):
                        op = merge_one(op, dv[s][0], dv[s][1], ms[s])
                    return op

                return lax.cond(hit, do_hits, lambda op: op, st)

            a0k, a0v, a1k, a1v, tau, flag = lax.fori_loop(
                0, (nchunk - 4) // 4, blk_body, st)

            kbuf_v[pl.ds(0, 16)] = a0k
            kbuf_v[pl.ds(16, 16)] = a1k
            g1 = plsc.load_gather(kbuf_v, [iota16 + 1])
            g2 = plsc.load_gather(kbuf_v, [iota16 + 17])
            tied = flag | jnp.any(a0k == g1) | jnp.any(a1k == g2)

            def repair(_):
                bigi = jnp.int32(2 ** 30)
                o0 = jnp.zeros((16,), jnp.int32)
                o1 = jnp.zeros((16,), jnp.int32)
                lastk = jnp.float32(-1e30)
                lasti = jnp.int32(-1)
                for s in range(KS):
                    def scan_chunk(j, acc):
                        bk, bi = acc
                        dj, ij = dchunk(j)
                        m = (dj > lastk) | ((dj == lastk) & (ij > lasti))
                        ck = jnp.where(m, dj, _INF)
                        ci = jnp.where(m, ij, bigi)
                        lt = (ck < bk) | ((ck == bk) & (ci < bi))
                        return (jnp.where(lt, ck, bk), jnp.where(lt, ci, bi))

                    bk, bi = lax.fori_loop(
                        0, nchunk, scan_chunk,
                        (jnp.full((16,), _INF, jnp.float32),
                         jnp.full((16,), bigi, jnp.int32)))
                    mk = jnp.min(bk)
                    mi = jnp.min(jnp.where(bk == mk, bi, bigi))
                    lastk = mk
                    lasti = mi
                    if s < 16:
                        o0 = jnp.where(iota16 == s, mi, o0)
                    else:
                        o1 = jnp.where(iota16 == (s - 16), mi, o1)
                return o0, o1

            o0, o1 = lax.cond(tied, repair, lambda _: (a0v, a1v), 0)
            idxb_v[g, pl.ds(0, 16)] = o0
            idxb_v[g, pl.ds(16, 16)] = o1
            return 0

        lax.fori_loop(0, g_per_w, row_body, 0)
        pltpu.sync_copy(idxb_v, out_hbm.at[pl.ds(wid * g_per_w, g_per_w)])

    return k(xt, crows)


def kernel(xyz, color):
    xt = xyz.transpose(2, 0, 1)
    fps_idx = _fps(xt)

    tab = jnp.concatenate(
        [xyz, color, jnp.zeros((B, N, TW - 9), jnp.float32)], axis=-1
    ).reshape(B * N, TW)
    bbase = (jnp.arange(B, dtype=jnp.int32) * N)[:, None]
    cidx = (fps_idx + bbase).reshape(-1)
    crows = _center_gather(tab, cidx)
    center = crows.reshape(B, NG, TW)[..., :3]

    idx = _knn_topk(xt, crows).reshape(B, NG, KS)

    nidx = (idx + bbase[:, :, None]).reshape(-1)
    nrows = _nbr_gather(tab, nidx, crows).reshape(B, NG, KS, TW)
    neighborhood = nrows[..., :3]
    features = nrows[..., :9]
    return (neighborhood, center, features)
